# Initial kernel scaffold; baseline (speedup 1.0000x reference)
import jax
import jax.numpy as jnp
from jax.experimental import pallas as pl


def kernel(h, batch, cdr_mask, iface_mask, Wk, Wv, Wq, Wres, Wout, ln_kv_g, ln_kv_b, ln_q_g, ln_q_b, cdr_bias, iface_bias, logit_scale):
    raise NotImplementedError("write your pallas kernel here")


# speedup vs baseline: 23.4862x; 23.4862x over previous
import jax
import jax.numpy as jnp
import numpy as np
from jax.experimental import pallas as pl
from jax.experimental.pallas import tpu as pltpu

DIM = 128
HEADS = 4
HEAD_DIM = DIM // HEADS
NUM_SEG = 64
EPS = 1e-5
MEAN_RES_SCALE = 0.2
BLK = 2000
NEG = -1e30


def _pass_a(h_ref, brow_ref, bcol_ref, sum_ref, cnt_ref, max_ref):
    i = pl.program_id(0)

    @pl.when(i == 0)
    def _init():
        sum_ref[...] = jnp.zeros_like(sum_ref)
        cnt_ref[...] = jnp.zeros_like(cnt_ref)
        max_ref[...] = jnp.full_like(max_ref, -jnp.inf)

    h = h_ref[...]
    brow = brow_ref[0]
    bcol = bcol_ref[...]
    seg_c = jax.lax.broadcasted_iota(jnp.int32, (NUM_SEG, brow.shape[1]), 0)
    onehot_t = (seg_c == brow).astype(jnp.float32)
    sum_ref[...] += jnp.dot(onehot_t, h, preferred_element_type=jnp.float32)
    cnt_ref[...] += jnp.sum(onehot_t, axis=1, keepdims=True)

    s0 = bcol[0, 0]
    s1 = bcol[bcol.shape[0] - 1, 0]
    seg_rows = jax.lax.broadcasted_iota(jnp.int32, (NUM_SEG, 1), 0)

    def body(s, _):
        m = jnp.max(jnp.where(bcol == s, h, -jnp.inf), axis=0, keepdims=True)
        max_ref[...] = jnp.where(seg_rows == s,
                                 jnp.maximum(max_ref[...], m), max_ref[...])
        return 0

    jax.lax.fori_loop(s0, s1 + 1, body, 0)


def _mid(sum_ref, cnt_ref, max_ref, wqT_ref, lng_ref, lnb_ref, wresT_ref,
         q_ref, mres_ref):
    mean = sum_ref[...] / cnt_ref[...]
    mx = max_ref[...]
    qa = (jnp.dot(mean, wqT_ref[:DIM, :], preferred_element_type=jnp.float32)
          + jnp.dot(mx, wqT_ref[DIM:, :], preferred_element_type=jnp.float32))
    mu = jnp.mean(qa, axis=-1, keepdims=True)
    var = jnp.mean((qa - mu) ** 2, axis=-1, keepdims=True)
    q_ref[...] = (qa - mu) / jnp.sqrt(var + EPS) * lng_ref[...] + lnb_ref[...]
    mres_ref[...] = MEAN_RES_SCALE * jnp.dot(
        mean, wresT_ref[...], preferred_element_type=jnp.float32)


def _pass_b(h_ref, brow_ref, bcol_ref, cdr_ref, ifc_ref, q_ref, mres_ref,
            wkT_ref, wvT_ref, woutT_ref, lng_ref, lnb_ref, scal_ref,
            out_ref, m_scr, d_scr, acc_scr):
    i = pl.program_id(0)
    nblk = pl.num_programs(0)

    @pl.when(i == 0)
    def _init():
        m_scr[...] = jnp.full_like(m_scr, NEG)
        d_scr[...] = jnp.zeros_like(d_scr)
        acc_scr[...] = jnp.zeros_like(acc_scr)

    h = h_ref[...]
    mu = jnp.mean(h, axis=-1, keepdims=True)
    var = jnp.mean((h - mu) ** 2, axis=-1, keepdims=True)
    h_ln = (h - mu) / jnp.sqrt(var + EPS) * lng_ref[...] + lnb_ref[...]
    k = jnp.dot(h_ln, wkT_ref[...], preferred_element_type=jnp.float32)
    v = jnp.dot(h_ln, wvT_ref[...], preferred_element_type=jnp.float32)

    bcol = bcol_ref[...]
    brow = brow_ref[0]
    blk = bcol.shape[0]
    seg_r = jax.lax.broadcasted_iota(jnp.int32, (blk, NUM_SEG), 1)
    onehot = (bcol == seg_r).astype(jnp.float32)
    seg_c = jax.lax.broadcasted_iota(jnp.int32, (NUM_SEG, blk), 0)
    onehot_t = (seg_c == brow).astype(jnp.float32)

    sd = jax.lax.broadcasted_iota(jnp.int32, (DIM, 8), 0) // HEAD_DIM
    sh = jax.lax.broadcasted_iota(jnp.int32, (DIM, 8), 1)
    sel = (sd == sh).astype(jnp.float32)
    td = jax.lax.broadcasted_iota(jnp.int32, (8, DIM), 1) // HEAD_DIM
    th = jax.lax.broadcasted_iota(jnp.int32, (8, DIM), 0)
    sel_t = (td == th).astype(jnp.float32)

    qrows = jnp.dot(onehot, q_ref[...], preferred_element_type=jnp.float32)
    logit = jnp.dot(k * qrows, sel, preferred_element_type=jnp.float32)
    logit = logit * (scal_ref[0, 2] / np.sqrt(HEAD_DIM))
    logit = logit + scal_ref[0, 0] * cdr_ref[...] + scal_ref[0, 1] * ifc_ref[...]

    s0 = bcol[0, 0]
    s1 = bcol[blk - 1, 0]
    seg_rows8 = jax.lax.broadcasted_iota(jnp.int32, (NUM_SEG, 8), 0)

    def body(s, _):
        lmax = jnp.max(jnp.where(bcol == s, logit, NEG), axis=0, keepdims=True)
        rowm = seg_rows8 == s
        mo = m_scr[...]
        mn = jnp.where(rowm, jnp.maximum(mo, lmax), mo)
        scale = jnp.where(rowm, jnp.exp(mo - mn), 1.0)
        m_scr[...] = mn
        d_scr[...] = d_scr[...] * scale
        acc_scr[...] = acc_scr[...] * jnp.dot(
            scale, sel_t, preferred_element_type=jnp.float32)
        return 0

    jax.lax.fori_loop(s0, s1 + 1, body, 0)

    mrows = jnp.dot(onehot, m_scr[...], preferred_element_type=jnp.float32)
    w = jnp.exp(logit - mrows)
    d_scr[...] += jnp.dot(onehot_t, w, preferred_element_type=jnp.float32)
    wb = jnp.dot(w, sel_t, preferred_element_type=jnp.float32)
    acc_scr[...] += jnp.dot(onehot_t, wb * v,
                            preferred_element_type=jnp.float32)

    @pl.when(i == nblk - 1)
    def _fin():
        denom = jnp.dot(d_scr[...], sel_t, preferred_element_type=jnp.float32)
        g_attn = acc_scr[...] / denom
        out_ref[...] = jnp.dot(g_attn, woutT_ref[...],
                               preferred_element_type=jnp.float32) + mres_ref[...]


def kernel(h, batch, cdr_mask, iface_mask, Wk, Wv, Wq, Wres, Wout,
           ln_kv_g, ln_kv_b, ln_q_g, ln_q_b, cdr_bias, iface_bias, logit_scale):
    n = h.shape[0]
    grid = n // BLK
    assert grid * BLK == n

    batch = batch.astype(jnp.int32)
    brow = batch.reshape(grid, 1, BLK)
    bcol = batch.reshape(n, 1)
    cdrf = cdr_mask.astype(jnp.float32).reshape(n, 1)
    ifcf = iface_mask.astype(jnp.float32).reshape(n, 1)

    seg_block = pl.BlockSpec((NUM_SEG, DIM), lambda i: (0, 0))
    row_block = pl.BlockSpec((BLK, DIM), lambda i: (i, 0))
    brow_block = pl.BlockSpec((1, 1, BLK), lambda i: (i, 0, 0))
    col_block = pl.BlockSpec((BLK, 1), lambda i: (i, 0))
    vec_block = pl.BlockSpec((1, DIM), lambda i: (0, 0))

    seg_sum, seg_cnt, seg_max = pl.pallas_call(
        _pass_a,
        grid=(grid,),
        in_specs=[row_block, brow_block, col_block],
        out_specs=[seg_block, seg_block, seg_block],
        out_shape=[jax.ShapeDtypeStruct((NUM_SEG, DIM), jnp.float32)] * 3,
        compiler_params=pltpu.CompilerParams(
            dimension_semantics=("arbitrary",)),
    )(h, brow, bcol)

    q, mres = pl.pallas_call(
        _mid,
        in_specs=[pl.BlockSpec((NUM_SEG, DIM), lambda: (0, 0))] * 3
        + [pl.BlockSpec((2 * DIM, DIM), lambda: (0, 0)),
           pl.BlockSpec((1, DIM), lambda: (0, 0)),
           pl.BlockSpec((1, DIM), lambda: (0, 0)),
           pl.BlockSpec((DIM, DIM), lambda: (0, 0))],
        out_specs=[pl.BlockSpec((NUM_SEG, DIM), lambda: (0, 0))] * 2,
        out_shape=[jax.ShapeDtypeStruct((NUM_SEG, DIM), jnp.float32)] * 2,
    )(seg_sum, seg_cnt, seg_max, Wq.T, ln_q_g.reshape(1, DIM),
      ln_q_b.reshape(1, DIM), Wres.T)

    scal = jnp.stack([cdr_bias, iface_bias, logit_scale]).reshape(1, 3)

    out = pl.pallas_call(
        _pass_b,
        grid=(grid,),
        in_specs=[row_block, brow_block, col_block, col_block, col_block,
                  seg_block, seg_block,
                  pl.BlockSpec((DIM, DIM), lambda i: (0, 0)),
                  pl.BlockSpec((DIM, DIM), lambda i: (0, 0)),
                  pl.BlockSpec((DIM, DIM), lambda i: (0, 0)),
                  vec_block, vec_block,
                  pl.BlockSpec((1, 3), lambda i: (0, 0))],
        out_specs=seg_block,
        out_shape=jax.ShapeDtypeStruct((NUM_SEG, DIM), jnp.float32),
        scratch_shapes=[pltpu.VMEM((NUM_SEG, 8), jnp.float32),
                        pltpu.VMEM((NUM_SEG, 8), jnp.float32),
                        pltpu.VMEM((NUM_SEG, DIM), jnp.float32)],
        compiler_params=pltpu.CompilerParams(
            dimension_semantics=("arbitrary",)),
    )(h, brow, bcol, cdrf, ifcf, q, mres, Wk.T, Wv.T, Wout.T,
      ln_kv_g.reshape(1, DIM), ln_kv_b.reshape(1, DIM), scal)

    return out


# speedup vs baseline: 30.8824x; 1.3149x over previous
import jax
import jax.numpy as jnp
import numpy as np
from jax.experimental import pallas as pl
from jax.experimental.pallas import tpu as pltpu

DIM = 128
HEADS = 4
HEAD_DIM = DIM // HEADS
NUM_SEG = 64
EPS = 1e-5
MEAN_RES_SCALE = 0.2
BLK = 2000
NEG = -1e30


def _fused(h_ref, brow_ref, bcol_ref, ci_ref, wqT_ref, lnqg_ref, lnqb_ref,
           wresT_ref, wkvT_ref, woutT_ref, lng_ref, lnb_ref, scal_ref,
           out_ref, sum_s, cnt_s, max_s, q_s, mres_s, m_s, d_s, acc_s):
    p = pl.program_id(0)
    i = pl.program_id(1)
    nblk = pl.num_programs(1)
    h = h_ref[...]
    bcol = bcol_ref[...]
    brow = brow_ref[0]
    blk = bcol.shape[0]
    seg_c = jax.lax.broadcasted_iota(jnp.int32, (NUM_SEG, blk), 0)
    onehot_t = (seg_c == brow).astype(jnp.float32)
    s0 = bcol[0, 0]
    s1 = bcol[blk - 1, 0]

    @pl.when(p == 0)
    def _phase_a():
        @pl.when(i == 0)
        def _init():
            sum_s[...] = jnp.zeros_like(sum_s)
            cnt_s[...] = jnp.zeros_like(cnt_s)
            max_s[...] = jnp.full_like(max_s, -jnp.inf)

        sum_s[...] += jnp.dot(onehot_t, h, preferred_element_type=jnp.float32)
        cnt_s[...] += jnp.sum(onehot_t, axis=1, keepdims=True)

        seg_rows = jax.lax.broadcasted_iota(jnp.int32, (NUM_SEG, 1), 0)

        def body(s, _):
            m = jnp.max(jnp.where(bcol == s, h, -jnp.inf), axis=0,
                        keepdims=True)
            max_s[...] = jnp.where(seg_rows == s,
                                   jnp.maximum(max_s[...], m), max_s[...])
            return 0

        jax.lax.fori_loop(s0, s1 + 1, body, 0)

    @pl.when(p == 1)
    def _phase_b():
        @pl.when(i == 0)
        def _mid():
            mean = sum_s[...] / cnt_s[...]
            qa = (jnp.dot(mean, wqT_ref[:DIM, :],
                          preferred_element_type=jnp.float32)
                  + jnp.dot(max_s[...], wqT_ref[DIM:, :],
                            preferred_element_type=jnp.float32))
            mu = jnp.mean(qa, axis=-1, keepdims=True)
            var = jnp.mean((qa - mu) ** 2, axis=-1, keepdims=True)
            q_s[...] = ((qa - mu) / jnp.sqrt(var + EPS) * lnqg_ref[...]
                        + lnqb_ref[...])
            mres_s[...] = MEAN_RES_SCALE * jnp.dot(
                mean, wresT_ref[...], preferred_element_type=jnp.float32)
            m_s[...] = jnp.full_like(m_s, NEG)
            d_s[...] = jnp.zeros_like(d_s)
            acc_s[...] = jnp.zeros_like(acc_s)

        r = jax.lax.broadcasted_iota(jnp.int32, (DIM, DIM), 0)
        c = jax.lax.broadcasted_iota(jnp.int32, (DIM, DIM), 1)
        cmat = (r == c).astype(jnp.float32) - (1.0 / DIM)
        h_c = jnp.dot(h, cmat, preferred_element_type=jnp.float32)
        msq = jnp.dot(h * h, jnp.full((DIM, DIM), 1.0 / DIM, jnp.float32),
                      preferred_element_type=jnp.float32)
        mu = jnp.dot(h, jnp.full((DIM, DIM), 1.0 / DIM, jnp.float32),
                     preferred_element_type=jnp.float32)
        var = msq - mu * mu
        h_ln = (h_c * jax.lax.rsqrt(var + EPS)) * lng_ref[...] + lnb_ref[...]

        kv = jnp.dot(h_ln, wkvT_ref[...], preferred_element_type=jnp.float32)
        k = kv[:, :DIM]
        v = kv[:, DIM:]

        seg_r = jax.lax.broadcasted_iota(jnp.int32, (blk, NUM_SEG), 1)
        onehot = (bcol == seg_r).astype(jnp.float32)
        qrows = jnp.dot(onehot, q_s[...], preferred_element_type=jnp.float32)

        sd = jax.lax.broadcasted_iota(jnp.int32, (DIM + 2, 8), 0)
        sh = jax.lax.broadcasted_iota(jnp.int32, (DIM + 2, 8), 1)
        lscale = scal_ref[0, 2] * (1.0 / np.sqrt(HEAD_DIM))
        sel_aug = jnp.where(sd // HEAD_DIM == sh, lscale, 0.0)
        sel_aug = jnp.where(sd == DIM, scal_ref[0, 0], sel_aug)
        sel_aug = jnp.where(sd == DIM + 1, scal_ref[0, 1], sel_aug)
        lhs = jnp.concatenate([k * qrows, ci_ref[...]], axis=1)
        logit = jnp.dot(lhs, sel_aug, preferred_element_type=jnp.float32)

        bmax = jnp.max(logit, axis=0, keepdims=True)
        seg8 = jax.lax.broadcasted_iota(jnp.int32, (NUM_SEG, 8), 0)
        present = (seg8 >= s0) & (seg8 <= s1)
        mo = m_s[...]
        mn = jnp.where(present, jnp.maximum(mo, bmax), mo)
        scale = jnp.exp(mo - mn)
        m_s[...] = mn

        td = jax.lax.broadcasted_iota(jnp.int32, (8, DIM), 1) // HEAD_DIM
        th = jax.lax.broadcasted_iota(jnp.int32, (8, DIM), 0)
        sel_t = (td == th).astype(jnp.float32)

        mrows = jnp.dot(onehot, mn, preferred_element_type=jnp.float32)
        w = jnp.exp(logit - mrows)
        d_s[...] = d_s[...] * scale + jnp.dot(
            onehot_t, w, preferred_element_type=jnp.float32)
        wb = jnp.dot(w, sel_t, preferred_element_type=jnp.float32)
        acc_s[...] = (acc_s[...] * jnp.dot(scale, sel_t,
                                           preferred_element_type=jnp.float32)
                      + jnp.dot(onehot_t, wb * v,
                                preferred_element_type=jnp.float32))

        @pl.when(i == nblk - 1)
        def _fin():
            denom = jnp.dot(d_s[...], sel_t,
                            preferred_element_type=jnp.float32)
            g_attn = acc_s[...] / denom
            out_ref[...] = jnp.dot(g_attn, woutT_ref[...],
                                   preferred_element_type=jnp.float32) \
                + mres_s[...]


def kernel(h, batch, cdr_mask, iface_mask, Wk, Wv, Wq, Wres, Wout,
           ln_kv_g, ln_kv_b, ln_q_g, ln_q_b, cdr_bias, iface_bias, logit_scale):
    n = h.shape[0]
    grid = n // BLK
    assert grid * BLK == n

    batch = batch.astype(jnp.int32)
    brow = batch.reshape(grid, 1, BLK)
    bcol = batch.reshape(n, 1)
    ci = jnp.stack([cdr_mask, iface_mask], axis=1).astype(jnp.float32)
    wkvT = jnp.concatenate([Wk.T, Wv.T], axis=1)
    scal = jnp.stack([cdr_bias, iface_bias, logit_scale]).reshape(1, 3)

    cmap = lambda p, i: (0, 0)
    out = pl.pallas_call(
        _fused,
        grid=(2, grid),
        in_specs=[pl.BlockSpec((BLK, DIM), lambda p, i: (i, 0)),
                  pl.BlockSpec((1, 1, BLK), lambda p, i: (i, 0, 0)),
                  pl.BlockSpec((BLK, 1), lambda p, i: (i, 0)),
                  pl.BlockSpec((BLK, 2), lambda p, i: (i, 0)),
                  pl.BlockSpec((2 * DIM, DIM), cmap),
                  pl.BlockSpec((1, DIM), cmap),
                  pl.BlockSpec((1, DIM), cmap),
                  pl.BlockSpec((DIM, DIM), cmap),
                  pl.BlockSpec((DIM, 2 * DIM), cmap),
                  pl.BlockSpec((DIM, DIM), cmap),
                  pl.BlockSpec((1, DIM), cmap),
                  pl.BlockSpec((1, DIM), cmap),
                  pl.BlockSpec((1, 3), cmap)],
        out_specs=pl.BlockSpec((NUM_SEG, DIM), cmap),
        out_shape=jax.ShapeDtypeStruct((NUM_SEG, DIM), jnp.float32),
        scratch_shapes=[pltpu.VMEM((NUM_SEG, DIM), jnp.float32),
                        pltpu.VMEM((NUM_SEG, DIM), jnp.float32),
                        pltpu.VMEM((NUM_SEG, DIM), jnp.float32),
                        pltpu.VMEM((NUM_SEG, DIM), jnp.float32),
                        pltpu.VMEM((NUM_SEG, DIM), jnp.float32),
                        pltpu.VMEM((NUM_SEG, 8), jnp.float32),
                        pltpu.VMEM((NUM_SEG, 8), jnp.float32),
                        pltpu.VMEM((NUM_SEG, DIM), jnp.float32)],
        compiler_params=pltpu.CompilerParams(
            dimension_semantics=("arbitrary", "arbitrary")),
    )(h, brow, bcol, ci, Wq.T, ln_q_g.reshape(1, DIM), ln_q_b.reshape(1, DIM),
      Wres.T, wkvT, Wout.T, ln_kv_g.reshape(1, DIM), ln_kv_b.reshape(1, DIM),
      scal)

    return out
